# baseline (speedup 1.0000x reference)
import functools

import jax
import jax.numpy as jnp
from jax import lax
from jax.experimental import pallas as pl
from jax.experimental.pallas import tpu as pltpu
from jax.experimental.pallas import tpu_sc as plsc

_B, _T, _E, _D = 16, 2048, 512, 512
_ROWS = _B * _T
_NW = 32
_RPW = _ROWS // _NW
_R = 16
_NB = 4
_NBO = 2
_NCHUNK = _RPW // _R
_LANES = 16
_NTOK = 1000


def _tables_body(tok_ref, sty_ref, w_ref, b_ref, ttst_ref):
    w = w_ref[...]
    ttst_ref[: _NTOK, :] = (
        jnp.dot(tok_ref[...], w, preferred_element_type=jnp.float32) + b_ref[...]
    )
    ttst_ref[_NTOK:, :] = jnp.dot(sty_ref[...], w,
                                  preferred_element_type=jnp.float32)


def _fused_tables(token_table, style_table, W, b):
    return pl.pallas_call(
        _tables_body,
        out_shape=jax.ShapeDtypeStruct((_NTOK + 256, _D), jnp.float32),
    )(token_table, style_table, W, b.reshape(1, _D))


def _sc_body(cidx_hbm, audio_hbm, noizy_hbm, times_hbm, ttst_hbm,
             out_hbm, cidx, timesv, noizyb, audb, gatb, outb, insems, outsems):
    wid = lax.axis_index("s") * 2 + lax.axis_index("c")
    base0 = wid * _RPW
    b_idx = wid // 2

    pltpu.sync_copy(cidx_hbm.at[pl.ds(base0 * 2, _RPW * 2)], cidx)
    pltpu.sync_copy(times_hbm.at[b_idx], timesv)
    tvec = timesv[...]

    def issue_in(ci, s):
        base = base0 + ci * _R
        pltpu.async_copy(ttst_hbm.at[cidx.at[pl.ds(ci * 2 * _R, 2 * _R)]],
                         gatb.at[s], insems[s])
        pltpu.async_copy(noizy_hbm.at[pl.ds(base, _R)], noizyb.at[s], insems[s])
        pltpu.async_copy(audio_hbm.at[pl.ds(base, _R)], audb.at[s], insems[s])

    def wait_in(ci, s):
        base = base0 + ci * _R
        pltpu.make_async_copy(ttst_hbm.at[cidx.at[pl.ds(ci * 2 * _R, 2 * _R)]],
                              gatb.at[s], insems[s]).wait()
        pltpu.make_async_copy(noizy_hbm.at[pl.ds(base, _R)], noizyb.at[s],
                              insems[s]).wait()
        pltpu.make_async_copy(audio_hbm.at[pl.ds(base, _R)], audb.at[s],
                              insems[s]).wait()

    def issue_out(ci, so):
        base = base0 + ci * _R
        pltpu.async_copy(outb.at[so], out_hbm.at[pl.ds(base, _R)], outsems[so])

    def wait_out(ci, so):
        base = base0 + ci * _R
        pltpu.make_async_copy(outb.at[so], out_hbm.at[pl.ds(base, _R)],
                              outsems[so]).wait()

    def compute(s, so):
        def row(r, c2):
            for kb in range(_D // 32):
                slm = pl.ds(kb * _LANES, _LANES)
                tw = gatb[s, r, slm]
                sw = gatb[s, _R + r, slm]
                t0 = lax.bitcast_convert_type(lax.shift_left(tw, 16), jnp.float32)
                t1 = lax.bitcast_convert_type(
                    lax.bitwise_and(tw, jnp.int32(-65536)), jnp.float32)
                s0 = lax.bitcast_convert_type(lax.shift_left(sw, 16), jnp.float32)
                s1 = lax.bitcast_convert_type(
                    lax.bitwise_and(sw, jnp.int32(-65536)), jnp.float32)
                sl0 = pl.ds(kb * 32, _LANES)
                sl1 = pl.ds(kb * 32 + _LANES, _LANES)
                outb[so, r, sl0] = (
                    noizyb[s, r, sl0] + t0 + s0 + tvec * audb[s, r, sl0]
                )
                outb[so, r, sl1] = (
                    noizyb[s, r, sl1] + t1 + s1 + tvec * audb[s, r, sl1]
                )
            return c2

        lax.fori_loop(0, _R, row, 0)

    for p in range(_NB - 1):
        issue_in(p, p)

    @pl.loop(0, _NCHUNK, step=_NB)
    def outer(ci0):
        for s in range(_NB):
            ci = ci0 + s
            so = s % _NBO
            wait_in(ci, s)
            ns = (s + _NB - 1) % _NB

            @pl.when(ci + _NB - 1 < _NCHUNK)
            def _():
                issue_in(ci + _NB - 1, ns)

            @pl.when(ci >= _NBO)
            def _():
                wait_out(ci - _NBO, so)

            compute(s, so)
            issue_out(ci, so)

    for ci in range(_NCHUNK - _NBO, _NCHUNK):
        wait_out(ci, ci % _NBO)


@functools.partial(jax.jit)
def kernel(tokens, tokens_style, audio, audio_noizy, times, token_table,
           style_table, W, b):
    ttst = _fused_tables(token_table, style_table, W, b)
    ttst = jnp.transpose(
        ttst.reshape(_NTOK + 256, _D // 32, 2, _LANES), (0, 1, 3, 2)
    ).reshape(_NTOK + 256, _D).astype(jnp.bfloat16)
    ttst = lax.bitcast_convert_type(
        ttst.reshape(_NTOK + 256, _D // 2, 2), jnp.int32
    )

    tok = tokens.reshape(_NW, _NCHUNK, _R).astype(jnp.int32)
    sty = tokens_style.reshape(_NW, _NCHUNK, _R).astype(jnp.int32) + _NTOK
    cidx = jnp.concatenate([tok, sty], axis=2).reshape(_ROWS * 2)

    mesh = plsc.VectorSubcoreMesh(core_axis_name="c", subcore_axis_name="s")
    sc = pl.kernel(
        _sc_body,
        out_type=jax.ShapeDtypeStruct((_ROWS, _D), jnp.float32),
        mesh=mesh,
        scratch_types=[
            pltpu.VMEM((_RPW * 2,), jnp.int32),
            pltpu.VMEM((_LANES,), jnp.float32),
            pltpu.VMEM((_NB, _R, _D), jnp.float32),
            pltpu.VMEM((_NB, _R, _D), jnp.float32),
            pltpu.VMEM((_NB, 2 * _R, _D // 2), jnp.int32),
            pltpu.VMEM((_NBO, _R, _D), jnp.float32),
            [pltpu.SemaphoreType.DMA] * _NB,
            [pltpu.SemaphoreType.DMA] * _NBO,
        ],
    )
    out = sc(
        cidx,
        audio.reshape(_ROWS, _D),
        audio_noizy.reshape(_ROWS, _D),
        jnp.broadcast_to(times[:, None], (_B, _LANES)),
        ttst,
    )
    return out.reshape(_B, _T, _D)


# speedup vs baseline: 1.1047x; 1.1047x over previous
import functools

import jax
import jax.numpy as jnp
from jax import lax
from jax.experimental import pallas as pl
from jax.experimental.pallas import tpu as pltpu
from jax.experimental.pallas import tpu_sc as plsc

_B, _T, _E, _D = 16, 2048, 512, 512
_ROWS = _B * _T
_NW = 32
_RPW = _ROWS // _NW
_R = 16
_NB = 2
_NCHUNK = _RPW // _R
_LANES = 16
_NTOK = 1000


def _tables_body(tok_ref, sty_ref, w_ref, b_ref, ttst_ref):
    w = w_ref[...]
    ttst_ref[: _NTOK, :] = (
        jnp.dot(tok_ref[...], w, preferred_element_type=jnp.float32) + b_ref[...]
    )
    ttst_ref[_NTOK:, :] = jnp.dot(sty_ref[...], w,
                                  preferred_element_type=jnp.float32)


def _fused_tables(token_table, style_table, W, b):
    return pl.pallas_call(
        _tables_body,
        out_shape=jax.ShapeDtypeStruct((_NTOK + 256, _D), jnp.float32),
    )(token_table, style_table, W, b.reshape(1, _D))


def _sc_body(cidx_hbm, audio_hbm, noizy_hbm, times_hbm, ttst_hbm,
             out_hbm, cidx, timesv, noizyb, audb, gatb, outb, insems, outsems):
    wid = lax.axis_index("s") * 2 + lax.axis_index("c")
    base0 = wid * _RPW
    b_idx = wid // 2

    pltpu.sync_copy(cidx_hbm.at[pl.ds(base0 * 2, _RPW * 2)], cidx)
    pltpu.sync_copy(times_hbm.at[b_idx], timesv)
    tvec = timesv[...]

    def issue_in(ci, s):
        base = base0 + ci * _R
        pltpu.async_copy(ttst_hbm.at[cidx.at[pl.ds(ci * 2 * _R, 2 * _R)]],
                         gatb.at[s], insems[s])
        pltpu.async_copy(noizy_hbm.at[pl.ds(base, _R)], noizyb.at[s], insems[s])
        pltpu.async_copy(audio_hbm.at[pl.ds(base, _R)], audb.at[s], insems[s])

    def wait_in(ci, s):
        base = base0 + ci * _R
        pltpu.make_async_copy(ttst_hbm.at[cidx.at[pl.ds(ci * 2 * _R, 2 * _R)]],
                              gatb.at[s], insems[s]).wait()
        pltpu.make_async_copy(noizy_hbm.at[pl.ds(base, _R)], noizyb.at[s],
                              insems[s]).wait()
        pltpu.make_async_copy(audio_hbm.at[pl.ds(base, _R)], audb.at[s],
                              insems[s]).wait()

    def issue_out(ci, s):
        base = base0 + ci * _R
        pltpu.async_copy(outb.at[s], out_hbm.at[pl.ds(base, _R)], outsems[s])

    def wait_out(ci, s):
        base = base0 + ci * _R
        pltpu.make_async_copy(outb.at[s], out_hbm.at[pl.ds(base, _R)],
                              outsems[s]).wait()

    def compute(s):
        @plsc.parallel_loop(0, _R, unroll=2)
        def row(r):
            for k in range(_D // _LANES):
                sl = pl.ds(k * _LANES, _LANES)
                outb[s, r, sl] = (
                    noizyb[s, r, sl] + gatb[s, r, sl] + gatb[s, _R + r, sl]
                    + tvec * audb[s, r, sl]
                )

    for p in range(_NB - 1):
        issue_in(p, p)

    @pl.loop(0, _NCHUNK, step=_NB)
    def outer(ci0):
        for s in range(_NB):
            ci = ci0 + s
            wait_in(ci, s)
            ns = (s + _NB - 1) % _NB

            @pl.when(ci + _NB - 1 < _NCHUNK)
            def _():
                issue_in(ci + _NB - 1, ns)

            @pl.when(ci >= _NB)
            def _():
                wait_out(ci - _NB, s)

            compute(s)
            issue_out(ci, s)

    for ci in range(_NCHUNK - _NB, _NCHUNK):
        wait_out(ci, ci % _NB)


@functools.partial(jax.jit)
def kernel(tokens, tokens_style, audio, audio_noizy, times, token_table,
           style_table, W, b):
    ttst = _fused_tables(token_table, style_table, W, b)

    tok = tokens.reshape(_NW, _NCHUNK, _R).astype(jnp.int32)
    sty = tokens_style.reshape(_NW, _NCHUNK, _R).astype(jnp.int32) + _NTOK
    cidx = jnp.concatenate([tok, sty], axis=2).reshape(_ROWS * 2)

    mesh = plsc.VectorSubcoreMesh(core_axis_name="c", subcore_axis_name="s")
    sc = pl.kernel(
        _sc_body,
        out_type=jax.ShapeDtypeStruct((_ROWS, _D), jnp.float32),
        mesh=mesh,
        scratch_types=[
            pltpu.VMEM((_RPW * 2,), jnp.int32),
            pltpu.VMEM((_LANES,), jnp.float32),
            pltpu.VMEM((_NB, _R, _D), jnp.float32),
            pltpu.VMEM((_NB, _R, _D), jnp.float32),
            pltpu.VMEM((_NB, 2 * _R, _D), jnp.float32),
            pltpu.VMEM((_NB, _R, _D), jnp.float32),
            [pltpu.SemaphoreType.DMA] * _NB,
            [pltpu.SemaphoreType.DMA] * _NB,
        ],
    )
    out = sc(
        cidx,
        audio.reshape(_ROWS, _D),
        audio_noizy.reshape(_ROWS, _D),
        jnp.broadcast_to(times[:, None], (_B, _LANES)),
        ttst,
    )
    return out.reshape(_B, _T, _D)


# speedup vs baseline: 1.3715x; 1.2415x over previous
import functools

import jax
import jax.numpy as jnp
from jax import lax
from jax.experimental import pallas as pl
from jax.experimental.pallas import tpu as pltpu
from jax.experimental.pallas import tpu_sc as plsc

_B, _T, _E, _D = 16, 2048, 512, 512
_ROWS = _B * _T
_NW = 32
_RPW = _ROWS // _NW
_R = 16
_NB = 2
_NBO = 2
_NCHUNK = _RPW // _R
_LANES = 16
_NTOK = 1000


def _tables_body(tok_ref, sty_ref, w_ref, b_ref, ttst_ref):
    w = w_ref[...]
    ttst_ref[: _NTOK, :] = (
        jnp.dot(tok_ref[...], w, preferred_element_type=jnp.float32) + b_ref[...]
    )
    ttst_ref[_NTOK:, :] = jnp.dot(sty_ref[...], w,
                                  preferred_element_type=jnp.float32)


def _fused_tables(token_table, style_table, W, b):
    return pl.pallas_call(
        _tables_body,
        out_shape=jax.ShapeDtypeStruct((_NTOK + 256, _D), jnp.float32),
    )(token_table, style_table, W, b.reshape(1, _D))


def _sc_body(cidx_hbm, audio_hbm, noizy_hbm, times_hbm, ttst_hbm,
             out_hbm, cidx, timesv, noizyb, audb, gatb, outb, insems, outsems):
    wid = lax.axis_index("s") * 2 + lax.axis_index("c")
    base0 = wid * _RPW
    b_idx = wid // 2

    pltpu.sync_copy(cidx_hbm.at[pl.ds(base0 * 2, _RPW * 2)], cidx)
    pltpu.sync_copy(times_hbm.at[b_idx], timesv)
    tvec = timesv[...]

    def issue_in(ci, s):
        base = base0 + ci * _R
        pltpu.async_copy(ttst_hbm.at[cidx.at[pl.ds(ci * 2 * _R, 2 * _R)]],
                         gatb.at[s], insems[s])
        pltpu.async_copy(noizy_hbm.at[pl.ds(base, _R)], noizyb.at[s], insems[s])
        pltpu.async_copy(audio_hbm.at[pl.ds(base, _R)], audb.at[s], insems[s])

    def wait_in(ci, s):
        base = base0 + ci * _R
        pltpu.make_async_copy(ttst_hbm.at[cidx.at[pl.ds(ci * 2 * _R, 2 * _R)]],
                              gatb.at[s], insems[s]).wait()
        pltpu.make_async_copy(noizy_hbm.at[pl.ds(base, _R)], noizyb.at[s],
                              insems[s]).wait()
        pltpu.make_async_copy(audio_hbm.at[pl.ds(base, _R)], audb.at[s],
                              insems[s]).wait()

    def issue_out(ci, so):
        base = base0 + ci * _R
        pltpu.async_copy(outb.at[so], out_hbm.at[pl.ds(base, _R)], outsems[so])

    def wait_out(ci, so):
        base = base0 + ci * _R
        pltpu.make_async_copy(outb.at[so], out_hbm.at[pl.ds(base, _R)],
                              outsems[so]).wait()

    def compute(s, so):
        @plsc.parallel_loop(0, _R, unroll=2)
        def row(r):
            for kb in range(_D // 32):
                slm = pl.ds(kb * _LANES, _LANES)
                tw = gatb[s, r, slm]
                sw = gatb[s, _R + r, slm]
                t0 = lax.bitcast_convert_type(lax.shift_left(tw, 16), jnp.float32)
                t1 = lax.bitcast_convert_type(
                    lax.bitwise_and(tw, jnp.int32(-65536)), jnp.float32)
                s0 = lax.bitcast_convert_type(lax.shift_left(sw, 16), jnp.float32)
                s1 = lax.bitcast_convert_type(
                    lax.bitwise_and(sw, jnp.int32(-65536)), jnp.float32)
                sl0 = pl.ds(kb * 32, _LANES)
                sl1 = pl.ds(kb * 32 + _LANES, _LANES)
                outb[so, r, sl0] = (
                    (noizyb[s, r, sl0] + t0)
                    + (s0 + tvec * audb[s, r, sl0])
                )
                outb[so, r, sl1] = (
                    (noizyb[s, r, sl1] + t1)
                    + (s1 + tvec * audb[s, r, sl1])
                )

    for p in range(_NB - 1):
        issue_in(p, p)

    @pl.loop(0, _NCHUNK, step=_NB)
    def outer(ci0):
        for s in range(_NB):
            ci = ci0 + s
            so = s % _NBO
            wait_in(ci, s)
            ns = (s + _NB - 1) % _NB

            @pl.when(ci + _NB - 1 < _NCHUNK)
            def _():
                issue_in(ci + _NB - 1, ns)

            @pl.when(ci >= _NBO)
            def _():
                wait_out(ci - _NBO, so)

            compute(s, so)
            issue_out(ci, so)

    for ci in range(_NCHUNK - _NBO, _NCHUNK):
        wait_out(ci, ci % _NBO)


@functools.partial(jax.jit)
def kernel(tokens, tokens_style, audio, audio_noizy, times, token_table,
           style_table, W, b):
    ttst = _fused_tables(token_table, style_table, W, b)
    ttst = jnp.transpose(
        ttst.reshape(_NTOK + 256, _D // 32, 2, _LANES), (0, 1, 3, 2)
    ).reshape(_NTOK + 256, _D).astype(jnp.bfloat16)
    ttst = lax.bitcast_convert_type(
        ttst.reshape(_NTOK + 256, _D // 2, 2), jnp.int32
    )

    tok = tokens.reshape(_NW, _NCHUNK, _R).astype(jnp.int32)
    sty = tokens_style.reshape(_NW, _NCHUNK, _R).astype(jnp.int32) + _NTOK
    cidx = jnp.concatenate([tok, sty], axis=2).reshape(_ROWS * 2)

    mesh = plsc.VectorSubcoreMesh(core_axis_name="c", subcore_axis_name="s")
    sc = pl.kernel(
        _sc_body,
        out_type=jax.ShapeDtypeStruct((_ROWS, _D), jnp.float32),
        mesh=mesh,
        scratch_types=[
            pltpu.VMEM((_RPW * 2,), jnp.int32),
            pltpu.VMEM((_LANES,), jnp.float32),
            pltpu.VMEM((_NB, _R, _D), jnp.float32),
            pltpu.VMEM((_NB, _R, _D), jnp.float32),
            pltpu.VMEM((_NB, 2 * _R, _D // 2), jnp.int32),
            pltpu.VMEM((_NBO, _R, _D), jnp.float32),
            [pltpu.SemaphoreType.DMA] * _NB,
            [pltpu.SemaphoreType.DMA] * _NBO,
        ],
    )
    out = sc(
        cidx,
        audio.reshape(_ROWS, _D),
        audio_noizy.reshape(_ROWS, _D),
        jnp.broadcast_to(times[:, None], (_B, _LANES)),
        ttst,
    )
    return out.reshape(_B, _T, _D)
